# Initial kernel scaffold; baseline (speedup 1.0000x reference)
import jax
import jax.numpy as jnp
from jax.experimental import pallas as pl


def kernel(image, label_image, label_costs):
    raise NotImplementedError("write your pallas kernel here")


# speedup vs baseline: 2.5764x; 2.5764x over previous
import jax
import jax.numpy as jnp
from jax.experimental import pallas as pl
from jax.experimental.pallas import tpu as pltpu

_CROP_H, _CROP_W = 720, 1280
_H, _W = 2160, 3840
_SAMPLES = 4
_IGNORE = 0
_NLAB = 20
_EPS = 1e-12

_PAD_H = _CROP_H + 8
_PAD_W = _CROP_W + 128
_MAX_RS = _H - _PAD_H
_MAX_CS = _W - _PAD_W


def _aligned_start(top, left):
    rs = jnp.minimum((top // 8) * 8, _MAX_RS)
    cs = jnp.minimum((left // 128) * 128, _MAX_CS)
    return rs, cs


def _hist_kernel(tops_ref, lefts_ref, label_hbm, hist_ref, buf, sem):
    s = pl.program_id(0)
    top = tops_ref[s]
    left = lefts_ref[s]
    rs = jnp.minimum((top // 8) * 8, _MAX_RS)
    cs = jnp.minimum((left // 128) * 128, _MAX_CS)
    cp = pltpu.make_async_copy(
        label_hbm.at[0, pl.ds(rs, _PAD_H), pl.ds(cs, _PAD_W)], buf, sem
    )
    cp.start()
    cp.wait()
    roff = top - rs
    coff = left - cs
    rows = jax.lax.broadcasted_iota(jnp.int32, (_PAD_H, _PAD_W), 0)
    cols = jax.lax.broadcasted_iota(jnp.int32, (_PAD_H, _PAD_W), 1)
    inside = (
        (rows >= roff)
        & (rows < roff + _CROP_H)
        & (cols >= coff)
        & (cols < coff + _CROP_W)
    )
    labels = jnp.where(inside, buf[...], 0)
    lane = jax.lax.broadcasted_iota(jnp.int32, (1, 128), 1)
    vec = jnp.zeros((1, 128), jnp.int32)
    for b in range(1, _NLAB):
        cnt = jnp.sum((labels == b).astype(jnp.int32))
        vec = vec + jnp.where(lane == b, cnt, 0)
    hist_ref[s] = vec


def _extract_kernel(off_ref, img_hbm, lab_hbm, img_out, lab_out, fbuf, ibuf,
                    sem_i, sem_l):
    c = pl.program_id(0)
    top = off_ref[0]
    left = off_ref[1]
    rs = jnp.minimum((top // 8) * 8, _MAX_RS)
    cs = jnp.minimum((left // 128) * 128, _MAX_CS)
    roff = top - rs
    coff = left - cs
    shift_r = (_PAD_H - roff) % _PAD_H
    shift_c = (_PAD_W - coff) % _PAD_W

    ci = pltpu.make_async_copy(
        img_hbm.at[c, pl.ds(rs, _PAD_H), pl.ds(cs, _PAD_W)], fbuf, sem_i
    )
    ci.start()

    @pl.when(c == 0)
    def _():
        cl = pltpu.make_async_copy(
            lab_hbm.at[0, pl.ds(rs, _PAD_H), pl.ds(cs, _PAD_W)], ibuf, sem_l
        )
        cl.start()
        cl.wait()
        v = pltpu.roll(ibuf[...], shift_r, 0)
        v = pltpu.roll(v, shift_c, 1)
        lab_out[0] = v[:_CROP_H, :_CROP_W]

    ci.wait()
    v = pltpu.roll(fbuf[...], shift_r, 0)
    v = pltpu.roll(v, shift_c, 1)
    img_out[0] = v[:_CROP_H, :_CROP_W]


def kernel(image, label_image, label_costs):
    crop_key = jax.random.key(42)
    tops, lefts = [], []
    for i in range(_SAMPLES):
        kt, kl = jax.random.split(jax.random.fold_in(crop_key, i))
        tops.append(jax.random.randint(kt, (), 0, _H - _CROP_H + 1))
        lefts.append(jax.random.randint(kl, (), 0, _W - _CROP_W + 1))
    tops = jnp.stack(tops).astype(jnp.int32)
    lefts = jnp.stack(lefts).astype(jnp.int32)

    hist_pad = pl.pallas_call(
        _hist_kernel,
        grid_spec=pltpu.PrefetchScalarGridSpec(
            num_scalar_prefetch=2,
            grid=(_SAMPLES,),
            in_specs=[pl.BlockSpec(memory_space=pl.ANY)],
            out_specs=pl.BlockSpec(memory_space=pltpu.VMEM),
            scratch_shapes=[
                pltpu.VMEM((_PAD_H, _PAD_W), jnp.int32),
                pltpu.SemaphoreType.DMA,
            ],
        ),
        out_shape=jax.ShapeDtypeStruct((_SAMPLES, 1, 128), jnp.int32),
    )(tops, lefts, label_image)

    nc = label_costs / jnp.maximum(jnp.sum(jnp.abs(label_costs)), _EPS)
    costs = []
    for s in range(_SAMPLES):
        h = jax.lax.optimization_barrier(hist_pad[s, 0, :_NLAB])
        hist = h.astype(jnp.float32).at[_IGNORE].set(0.0)
        dist = hist / jnp.maximum(jnp.sum(jnp.abs(hist)), _EPS)
        costs.append(jnp.sum(nc * dist))
    best_cost = costs[0]
    best_top = tops[0]
    best_left = lefts[0]
    for i in range(1, _SAMPLES):
        better = costs[i] < best_cost
        best_cost = jnp.where(better, costs[i], best_cost)
        best_top = jnp.where(better, tops[i], best_top)
        best_left = jnp.where(better, lefts[i], best_left)

    offs = jnp.stack([best_top, best_left])
    best_image, best_label = pl.pallas_call(
        _extract_kernel,
        grid_spec=pltpu.PrefetchScalarGridSpec(
            num_scalar_prefetch=1,
            grid=(3,),
            in_specs=[
                pl.BlockSpec(memory_space=pl.ANY),
                pl.BlockSpec(memory_space=pl.ANY),
            ],
            out_specs=[
                pl.BlockSpec((1, _CROP_H, _CROP_W), lambda c, *_: (c, 0, 0)),
                pl.BlockSpec((1, _CROP_H, _CROP_W), lambda c, *_: (0, 0, 0)),
            ],
            scratch_shapes=[
                pltpu.VMEM((_PAD_H, _PAD_W), jnp.float32),
                pltpu.VMEM((_PAD_H, _PAD_W), jnp.int32),
                pltpu.SemaphoreType.DMA,
                pltpu.SemaphoreType.DMA,
            ],
        ),
        out_shape=[
            jax.ShapeDtypeStruct((3, _CROP_H, _CROP_W), jnp.float32),
            jax.ShapeDtypeStruct((1, _CROP_H, _CROP_W), jnp.int32),
        ],
    )(offs, image, label_image)

    return best_image, best_label, best_cost


# speedup vs baseline: 8.9846x; 3.4872x over previous
import jax
import jax.numpy as jnp
from jax.experimental import pallas as pl
from jax.experimental.pallas import tpu as pltpu

_CROP_H, _CROP_W = 720, 1280
_H, _W = 2160, 3840
_SAMPLES = 4
_IGNORE = 0
_NLAB = 20
_EPS = 1e-12

_PAD_H = _CROP_H + 8
_PAD_W = _CROP_W + 128
_MAX_RS = _H - _PAD_H
_MAX_CS = _W - _PAD_W


def _crop_offsets():
    crop_key = jax.random.key(42)
    tops, lefts = [], []
    for i in range(_SAMPLES):
        kt, kl = jax.random.split(jax.random.fold_in(crop_key, i))
        tops.append(int(jax.random.randint(kt, (), 0, _H - _CROP_H + 1)))
        lefts.append(int(jax.random.randint(kl, (), 0, _W - _CROP_W + 1)))
    return tuple(tops), tuple(lefts)


_TOPS, _LEFTS = _crop_offsets()


def _aligned(top, left):
    rs = min((top // 8) * 8, _MAX_RS)
    cs = min((left // 128) * 128, _MAX_CS)
    return rs, cs


def _hist_kernel(label_hbm, hist_ref, buf0, buf1, sem0, sem1):
    bufs = (buf0, buf1)
    sems = (sem0, sem1)

    def copy(s):
        rs, cs = _aligned(_TOPS[s], _LEFTS[s])
        return pltpu.make_async_copy(
            label_hbm.at[0, pl.ds(rs, _PAD_H), pl.ds(cs, _PAD_W)],
            bufs[s % 2],
            sems[s % 2],
        )

    copy(0).start()
    copy(1).start()
    for s in range(_SAMPLES):
        copy(s).wait()
        rs, cs = _aligned(_TOPS[s], _LEFTS[s])
        roff = _TOPS[s] - rs
        coff = _LEFTS[s] - cs
        labels = bufs[s % 2][roff : roff + _CROP_H, :]
        cols = jax.lax.broadcasted_iota(jnp.int32, (_CROP_H, _PAD_W), 1)
        inside = (cols >= coff) & (cols < coff + _CROP_W)
        labels = jnp.where(inside, labels, 0)
        lane = jax.lax.broadcasted_iota(jnp.int32, (1, 128), 1)
        vec = jnp.zeros((1, 128), jnp.int32)
        for b in range(1, _NLAB):
            cnt = jnp.sum((labels == b).astype(jnp.int32))
            vec = vec + jnp.where(lane == b, cnt, 0)
        hist_ref[s] = vec
        if s + 2 < _SAMPLES:
            copy(s + 2).start()


def _extract_kernel(off_ref, img_hbm, lab_hbm, img_out, lab_out, fbuf, ibuf,
                    sem_i, sem_l):
    c = pl.program_id(0)
    top = off_ref[0]
    left = off_ref[1]
    rs = jnp.minimum((top // 8) * 8, _MAX_RS)
    cs = jnp.minimum((left // 128) * 128, _MAX_CS)
    roff = top - rs
    coff = left - cs
    shift_r = (_PAD_H - roff) % _PAD_H
    shift_c = (_PAD_W - coff) % _PAD_W

    ci = pltpu.make_async_copy(
        img_hbm.at[c, pl.ds(rs, _PAD_H), pl.ds(cs, _PAD_W)], fbuf, sem_i
    )
    ci.start()

    @pl.when(c == 0)
    def _():
        cl = pltpu.make_async_copy(
            lab_hbm.at[0, pl.ds(rs, _PAD_H), pl.ds(cs, _PAD_W)], ibuf, sem_l
        )
        cl.start()
        cl.wait()
        v = pltpu.roll(ibuf[...], shift_r, 0)
        v = pltpu.roll(v, shift_c, 1)
        lab_out[0] = v[:_CROP_H, :_CROP_W]

    ci.wait()
    v = pltpu.roll(fbuf[...], shift_r, 0)
    v = pltpu.roll(v, shift_c, 1)
    img_out[0] = v[:_CROP_H, :_CROP_W]


def kernel(image, label_image, label_costs):
    tops = jnp.array(_TOPS, dtype=jnp.int32)
    lefts = jnp.array(_LEFTS, dtype=jnp.int32)

    hist_pad = pl.pallas_call(
        _hist_kernel,
        in_specs=[pl.BlockSpec(memory_space=pl.ANY)],
        out_specs=pl.BlockSpec(memory_space=pltpu.VMEM),
        scratch_shapes=[
            pltpu.VMEM((_PAD_H, _PAD_W), jnp.int32),
            pltpu.VMEM((_PAD_H, _PAD_W), jnp.int32),
            pltpu.SemaphoreType.DMA,
            pltpu.SemaphoreType.DMA,
        ],
        out_shape=jax.ShapeDtypeStruct((_SAMPLES, 1, 128), jnp.int32),
    )(label_image)

    nc = label_costs / jnp.maximum(jnp.sum(jnp.abs(label_costs)), _EPS)
    costs = []
    for s in range(_SAMPLES):
        h = jax.lax.optimization_barrier(hist_pad[s, 0, :_NLAB])
        hist = h.astype(jnp.float32).at[_IGNORE].set(0.0)
        dist = hist / jnp.maximum(jnp.sum(jnp.abs(hist)), _EPS)
        costs.append(jnp.sum(nc * dist))
    best_cost = costs[0]
    best_top = tops[0]
    best_left = lefts[0]
    for i in range(1, _SAMPLES):
        better = costs[i] < best_cost
        best_cost = jnp.where(better, costs[i], best_cost)
        best_top = jnp.where(better, tops[i], best_top)
        best_left = jnp.where(better, lefts[i], best_left)

    offs = jnp.stack([best_top, best_left])
    best_image, best_label = pl.pallas_call(
        _extract_kernel,
        grid_spec=pltpu.PrefetchScalarGridSpec(
            num_scalar_prefetch=1,
            grid=(3,),
            in_specs=[
                pl.BlockSpec(memory_space=pl.ANY),
                pl.BlockSpec(memory_space=pl.ANY),
            ],
            out_specs=[
                pl.BlockSpec((1, _CROP_H, _CROP_W), lambda c, *_: (c, 0, 0)),
                pl.BlockSpec((1, _CROP_H, _CROP_W), lambda c, *_: (0, 0, 0)),
            ],
            scratch_shapes=[
                pltpu.VMEM((_PAD_H, _PAD_W), jnp.float32),
                pltpu.VMEM((_PAD_H, _PAD_W), jnp.int32),
                pltpu.SemaphoreType.DMA,
                pltpu.SemaphoreType.DMA,
            ],
        ),
        out_shape=[
            jax.ShapeDtypeStruct((3, _CROP_H, _CROP_W), jnp.float32),
            jax.ShapeDtypeStruct((1, _CROP_H, _CROP_W), jnp.int32),
        ],
    )(offs, image, label_image)

    return best_image, best_label, best_cost


# speedup vs baseline: 10.5687x; 1.1763x over previous
import jax
import jax.numpy as jnp
from jax.experimental import pallas as pl
from jax.experimental.pallas import tpu as pltpu

_CROP_H, _CROP_W = 720, 1280
_H, _W = 2160, 3840
_SAMPLES = 4
_IGNORE = 0
_NLAB = 20
_EPS = 1e-12

_PAD_H = _CROP_H + 8
_PAD_W = _CROP_W + 128
_MAX_RS = _H - _PAD_H
_MAX_CS = _W - _PAD_W


def _crop_offsets():
    crop_key = jax.random.key(42)
    tops, lefts = [], []
    for i in range(_SAMPLES):
        kt, kl = jax.random.split(jax.random.fold_in(crop_key, i))
        tops.append(int(jax.random.randint(kt, (), 0, _H - _CROP_H + 1)))
        lefts.append(int(jax.random.randint(kl, (), 0, _W - _CROP_W + 1)))
    return tuple(tops), tuple(lefts)


_TOPS, _LEFTS = _crop_offsets()


def _aligned(top, left):
    rs = min((top // 8) * 8, _MAX_RS)
    cs = min((left // 128) * 128, _MAX_CS)
    return rs, cs


def _hist_kernel(label_hbm, hist_ref, buf0, buf1, sem0, sem1):
    bufs = (buf0, buf1)
    sems = (sem0, sem1)

    def copy(s):
        rs, cs = _aligned(_TOPS[s], _LEFTS[s])
        return pltpu.make_async_copy(
            label_hbm.at[0, pl.ds(rs, _PAD_H), pl.ds(cs, _PAD_W)],
            bufs[s % 2],
            sems[s % 2],
        )

    copy(0).start()
    copy(1).start()
    for s in range(_SAMPLES):
        copy(s).wait()
        rs, cs = _aligned(_TOPS[s], _LEFTS[s])
        roff = _TOPS[s] - rs
        coff = _LEFTS[s] - cs
        labels = bufs[s % 2][roff : roff + _CROP_H, :]
        cols = jax.lax.broadcasted_iota(jnp.int32, (_CROP_H, _PAD_W), 1)
        inside = (cols >= coff) & (cols < coff + _CROP_W)
        labels = jnp.where(inside, labels, 0)
        lane = jax.lax.broadcasted_iota(jnp.int32, (1, 128), 1)
        vec = jnp.zeros((1, 128), jnp.int32)
        for b in range(1, _NLAB):
            cnt = jnp.sum((labels == b).astype(jnp.int32))
            vec = vec + jnp.where(lane == b, cnt, 0)
        hist_ref[s] = vec
        if s + 2 < _SAMPLES:
            copy(s + 2).start()


def _make_extract_kernel(s):
    top, left = _TOPS[s], _LEFTS[s]
    rs, cs = _aligned(top, left)
    roff = top - rs
    coff = left - cs

    def _extract_kernel(img_hbm, lab_hbm, img_out, lab_out, fbuf0, fbuf1,
                        ibuf, sem0, sem1, sem_l):

        def icopy(c, buf, sem):
            return pltpu.make_async_copy(
                img_hbm.at[c, pl.ds(rs, _PAD_H), pl.ds(cs, _PAD_W)], buf, sem
            )

        c0 = icopy(0, fbuf0, sem0)
        c1 = icopy(1, fbuf1, sem1)
        cl = pltpu.make_async_copy(
            lab_hbm.at[0, pl.ds(rs, _PAD_H), pl.ds(cs, _PAD_W)], ibuf, sem_l
        )
        c0.start()
        c1.start()
        cl.start()
        c0.wait()
        img_out[0] = fbuf0[roff : roff + _CROP_H, coff : coff + _CROP_W]
        cl.wait()
        lab_out[0] = ibuf[roff : roff + _CROP_H, coff : coff + _CROP_W]
        c1.wait()
        img_out[1] = fbuf1[roff : roff + _CROP_H, coff : coff + _CROP_W]
        c2 = icopy(2, fbuf0, sem0)
        c2.start()
        c2.wait()
        img_out[2] = fbuf0[roff : roff + _CROP_H, coff : coff + _CROP_W]

    return _extract_kernel


def kernel(image, label_image, label_costs):
    hist_pad = pl.pallas_call(
        _hist_kernel,
        in_specs=[pl.BlockSpec(memory_space=pl.ANY)],
        out_specs=pl.BlockSpec(memory_space=pltpu.VMEM),
        scratch_shapes=[
            pltpu.VMEM((_PAD_H, _PAD_W), jnp.int32),
            pltpu.VMEM((_PAD_H, _PAD_W), jnp.int32),
            pltpu.SemaphoreType.DMA,
            pltpu.SemaphoreType.DMA,
        ],
        out_shape=jax.ShapeDtypeStruct((_SAMPLES, 1, 128), jnp.int32),
    )(label_image)

    nc = label_costs / jnp.maximum(jnp.sum(jnp.abs(label_costs)), _EPS)
    costs = []
    for s in range(_SAMPLES):
        h = jax.lax.optimization_barrier(hist_pad[s, 0, :_NLAB])
        hist = h.astype(jnp.float32).at[_IGNORE].set(0.0)
        dist = hist / jnp.maximum(jnp.sum(jnp.abs(hist)), _EPS)
        costs.append(jnp.sum(nc * dist))
    best_cost = costs[0]
    best_idx = jnp.int32(0)
    for i in range(1, _SAMPLES):
        better = costs[i] < best_cost
        best_cost = jnp.where(better, costs[i], best_cost)
        best_idx = jnp.where(better, jnp.int32(i), best_idx)

    def _branch(s):
        def run(image, label_image):
            return pl.pallas_call(
                _make_extract_kernel(s),
                in_specs=[
                    pl.BlockSpec(memory_space=pl.ANY),
                    pl.BlockSpec(memory_space=pl.ANY),
                ],
                out_specs=[
                    pl.BlockSpec(memory_space=pltpu.VMEM),
                    pl.BlockSpec(memory_space=pltpu.VMEM),
                ],
                scratch_shapes=[
                    pltpu.VMEM((_PAD_H, _PAD_W), jnp.float32),
                    pltpu.VMEM((_PAD_H, _PAD_W), jnp.float32),
                    pltpu.VMEM((_PAD_H, _PAD_W), jnp.int32),
                    pltpu.SemaphoreType.DMA,
                    pltpu.SemaphoreType.DMA,
                    pltpu.SemaphoreType.DMA,
                ],
                out_shape=[
                    jax.ShapeDtypeStruct((3, _CROP_H, _CROP_W), jnp.float32),
                    jax.ShapeDtypeStruct((1, _CROP_H, _CROP_W), jnp.int32),
                ],
            )(image, label_image)

        return run

    best_image, best_label = jax.lax.switch(
        best_idx, [_branch(s) for s in range(_SAMPLES)], image, label_image
    )

    return best_image, best_label, best_cost


# speedup vs baseline: 11.0316x; 1.0438x over previous
import jax
import jax.numpy as jnp
from jax.experimental import pallas as pl
from jax.experimental.pallas import tpu as pltpu

_CROP_H, _CROP_W = 720, 1280
_H, _W = 2160, 3840
_SAMPLES = 4
_IGNORE = 0
_NLAB = 20
_EPS = 1e-12

_PAD_H = _CROP_H + 8
_PAD_W = _CROP_W + 128
_MAX_RS = _H - _PAD_H
_MAX_CS = _W - _PAD_W


def _crop_offsets():
    try:
        crop_key = jax.random.key(42)
        tops, lefts = [], []
        for i in range(_SAMPLES):
            kt, kl = jax.random.split(jax.random.fold_in(crop_key, i))
            tops.append(int(jax.random.randint(kt, (), 0, _H - _CROP_H + 1)))
            lefts.append(int(jax.random.randint(kl, (), 0, _W - _CROP_W + 1)))
        return tuple(tops), tuple(lefts)
    except Exception:
        return (564, 73, 133, 1175), (2217, 879, 2278, 255)


_TOPS, _LEFTS = _crop_offsets()


def _aligned(top, left):
    rs = min((top // 8) * 8, _MAX_RS)
    cs = min((left // 128) * 128, _MAX_CS)
    return rs, cs


def _hist_kernel(label_hbm, hist_ref, buf0, buf1, sem0, sem1):
    bufs = (buf0, buf1)
    sems = (sem0, sem1)

    def copy(s):
        rs, cs = _aligned(_TOPS[s], _LEFTS[s])
        return pltpu.make_async_copy(
            label_hbm.at[0, pl.ds(rs, _PAD_H), pl.ds(cs, _PAD_W)],
            bufs[s % 2],
            sems[s % 2],
        )

    copy(0).start()
    copy(1).start()
    for s in range(_SAMPLES):
        copy(s).wait()
        rs, cs = _aligned(_TOPS[s], _LEFTS[s])
        roff = _TOPS[s] - rs
        coff = _LEFTS[s] - cs
        labels = bufs[s % 2][roff : roff + _CROP_H, :]
        cols = jax.lax.broadcasted_iota(jnp.int32, (_CROP_H, _PAD_W), 1)
        inside = (cols >= coff) & (cols < coff + _CROP_W)
        labels = jnp.where(inside, labels, 0)
        sv = jnp.left_shift(jnp.int32(1), (labels & 3) * 8)
        g = labels >> 2
        parts = []
        for gi in range(5):
            contrib = jnp.where(g == gi, sv, 0)
            parts.append(contrib.reshape(3, 240, _PAD_W).sum(axis=1))
        lane = jax.lax.broadcasted_iota(jnp.int32, (1, 128), 1)
        vec = jnp.zeros((1, 128), jnp.int32)
        for b in range(1, _NLAB):
            cnt = jnp.sum((parts[b >> 2] >> (8 * (b & 3))) & 255)
            vec = vec + jnp.where(lane == b, cnt, 0)
        hist_ref[s] = vec
        if s + 2 < _SAMPLES:
            copy(s + 2).start()


def _make_extract_kernel(s):
    top, left = _TOPS[s], _LEFTS[s]
    rs, cs = _aligned(top, left)
    roff = top - rs
    coff = left - cs

    def _extract_kernel(img_hbm, lab_hbm, img_out, lab_out, fbuf0, fbuf1,
                        ibuf, sem0, sem1, sem_l):

        def icopy(c, buf, sem):
            return pltpu.make_async_copy(
                img_hbm.at[c, pl.ds(rs, _PAD_H), pl.ds(cs, _PAD_W)], buf, sem
            )

        c0 = icopy(0, fbuf0, sem0)
        c1 = icopy(1, fbuf1, sem1)
        cl = pltpu.make_async_copy(
            lab_hbm.at[0, pl.ds(rs, _PAD_H), pl.ds(cs, _PAD_W)], ibuf, sem_l
        )
        c0.start()
        c1.start()
        cl.start()
        c0.wait()
        img_out[0] = fbuf0[roff : roff + _CROP_H, coff : coff + _CROP_W]
        cl.wait()
        lab_out[0] = ibuf[roff : roff + _CROP_H, coff : coff + _CROP_W]
        c1.wait()
        img_out[1] = fbuf1[roff : roff + _CROP_H, coff : coff + _CROP_W]
        c2 = icopy(2, fbuf0, sem0)
        c2.start()
        c2.wait()
        img_out[2] = fbuf0[roff : roff + _CROP_H, coff : coff + _CROP_W]

    return _extract_kernel


def kernel(image, label_image, label_costs):
    hist_pad = pl.pallas_call(
        _hist_kernel,
        in_specs=[pl.BlockSpec(memory_space=pl.ANY)],
        out_specs=pl.BlockSpec(memory_space=pltpu.VMEM),
        scratch_shapes=[
            pltpu.VMEM((_PAD_H, _PAD_W), jnp.int32),
            pltpu.VMEM((_PAD_H, _PAD_W), jnp.int32),
            pltpu.SemaphoreType.DMA,
            pltpu.SemaphoreType.DMA,
        ],
        out_shape=jax.ShapeDtypeStruct((_SAMPLES, 1, 128), jnp.int32),
    )(label_image)

    nc = label_costs / jnp.maximum(jnp.sum(jnp.abs(label_costs)), _EPS)
    costs = []
    for s in range(_SAMPLES):
        h = jax.lax.optimization_barrier(hist_pad[s, 0, :_NLAB])
        hist = h.astype(jnp.float32).at[_IGNORE].set(0.0)
        dist = hist / jnp.maximum(jnp.sum(jnp.abs(hist)), _EPS)
        costs.append(jnp.sum(nc * dist))
    best_cost = costs[0]
    best_idx = jnp.int32(0)
    for i in range(1, _SAMPLES):
        better = costs[i] < best_cost
        best_cost = jnp.where(better, costs[i], best_cost)
        best_idx = jnp.where(better, jnp.int32(i), best_idx)

    def _branch(s):
        def run(image, label_image):
            return pl.pallas_call(
                _make_extract_kernel(s),
                in_specs=[
                    pl.BlockSpec(memory_space=pl.ANY),
                    pl.BlockSpec(memory_space=pl.ANY),
                ],
                out_specs=[
                    pl.BlockSpec(memory_space=pltpu.VMEM),
                    pl.BlockSpec(memory_space=pltpu.VMEM),
                ],
                scratch_shapes=[
                    pltpu.VMEM((_PAD_H, _PAD_W), jnp.float32),
                    pltpu.VMEM((_PAD_H, _PAD_W), jnp.float32),
                    pltpu.VMEM((_PAD_H, _PAD_W), jnp.int32),
                    pltpu.SemaphoreType.DMA,
                    pltpu.SemaphoreType.DMA,
                    pltpu.SemaphoreType.DMA,
                ],
                out_shape=[
                    jax.ShapeDtypeStruct((3, _CROP_H, _CROP_W), jnp.float32),
                    jax.ShapeDtypeStruct((1, _CROP_H, _CROP_W), jnp.int32),
                ],
            )(image, label_image)

        return run

    best_image, best_label = jax.lax.switch(
        best_idx, [_branch(s) for s in range(_SAMPLES)], image, label_image
    )

    return best_image, best_label, best_cost


# speedup vs baseline: 11.3032x; 1.0246x over previous
import jax
import jax.numpy as jnp
from jax.experimental import pallas as pl
from jax.experimental.pallas import tpu as pltpu

_CROP_H, _CROP_W = 720, 1280
_H, _W = 2160, 3840
_SAMPLES = 4
_IGNORE = 0
_NLAB = 20
_EPS = 1e-12

_PAD_H = _CROP_H + 8
_PAD_W = _CROP_W + 128
_MAX_RS = _H - _PAD_H
_MAX_CS = _W - _PAD_W


def _crop_offsets():
    try:
        crop_key = jax.random.key(42)
        tops, lefts = [], []
        for i in range(_SAMPLES):
            kt, kl = jax.random.split(jax.random.fold_in(crop_key, i))
            tops.append(int(jax.random.randint(kt, (), 0, _H - _CROP_H + 1)))
            lefts.append(int(jax.random.randint(kl, (), 0, _W - _CROP_W + 1)))
        return tuple(tops), tuple(lefts)
    except Exception:
        return (564, 73, 133, 1175), (2217, 879, 2278, 255)


_TOPS, _LEFTS = _crop_offsets()


def _aligned(top, left):
    rs = min((top // 8) * 8, _MAX_RS)
    cs = min((left // 128) * 128, _MAX_CS)
    return rs, cs


def _hist_kernel(label_hbm, hist_ref, buf0, buf1, sem0, sem1):
    bufs = (buf0, buf1)
    sems = (sem0, sem1)

    def copy(s):
        rs, cs = _aligned(_TOPS[s], _LEFTS[s])
        return pltpu.make_async_copy(
            label_hbm.at[0, pl.ds(rs, _PAD_H), pl.ds(cs, _PAD_W)],
            bufs[s % 2],
            sems[s % 2],
        )

    copy(0).start()
    copy(1).start()
    for s in range(_SAMPLES):
        copy(s).wait()
        rs, cs = _aligned(_TOPS[s], _LEFTS[s])
        roff = _TOPS[s] - rs
        coff = _LEFTS[s] - cs
        labels = bufs[s % 2][roff : roff + _CROP_H, :]
        cols = jax.lax.broadcasted_iota(jnp.int32, (_CROP_H, _PAD_W), 1)
        inside = (cols >= coff) & (cols < coff + _CROP_W)
        labels = jnp.where(inside, labels, 0)
        sv = jnp.left_shift(jnp.int32(1), (labels & 3) * 8)
        g = labels >> 2
        parts = []
        for gi in range(5):
            contrib = jnp.where(g == gi, sv, 0)
            parts.append(contrib.reshape(3, 240, _PAD_W).sum(axis=1))
        lane = jax.lax.broadcasted_iota(jnp.int32, (1, 128), 1)
        vec = jnp.zeros((1, 128), jnp.int32)
        total = jnp.int32(0)
        for b in range(1, _NLAB):
            cnt = jnp.sum((parts[b >> 2] >> (8 * (b & 3))) & 255)
            vec = vec + jnp.where(lane == b, cnt, 0)
            total = total + cnt
        fvec = vec.astype(jnp.float32)
        Sv = jnp.maximum(total.astype(jnp.float32), jnp.float32(_EPS))
        fvec = fvec + jnp.where(lane == _NLAB, Sv, 0.0)
        hist_ref[s] = fvec
        if s + 2 < _SAMPLES:
            copy(s + 2).start()


def _make_extract_kernel(s):
    top, left = _TOPS[s], _LEFTS[s]
    rs, cs = _aligned(top, left)
    roff = top - rs
    coff = left - cs

    def _extract_kernel(img_hbm, lab_hbm, img_out, lab_out, fbuf0, fbuf1,
                        ibuf, sem0, sem1, sem_l):

        def icopy(c, buf, sem):
            return pltpu.make_async_copy(
                img_hbm.at[c, pl.ds(rs, _PAD_H), pl.ds(cs, _PAD_W)], buf, sem
            )

        c0 = icopy(0, fbuf0, sem0)
        c1 = icopy(1, fbuf1, sem1)
        cl = pltpu.make_async_copy(
            lab_hbm.at[0, pl.ds(rs, _PAD_H), pl.ds(cs, _PAD_W)], ibuf, sem_l
        )
        c0.start()
        c1.start()
        cl.start()
        c0.wait()
        img_out[0] = fbuf0[roff : roff + _CROP_H, coff : coff + _CROP_W]
        cl.wait()
        lab_out[0] = ibuf[roff : roff + _CROP_H, coff : coff + _CROP_W]
        c1.wait()
        img_out[1] = fbuf1[roff : roff + _CROP_H, coff : coff + _CROP_W]
        c2 = icopy(2, fbuf0, sem0)
        c2.start()
        c2.wait()
        img_out[2] = fbuf0[roff : roff + _CROP_H, coff : coff + _CROP_W]

    return _extract_kernel


def kernel(image, label_image, label_costs):
    hist_pad = pl.pallas_call(
        _hist_kernel,
        in_specs=[pl.BlockSpec(memory_space=pl.ANY)],
        out_specs=pl.BlockSpec(memory_space=pltpu.VMEM),
        scratch_shapes=[
            pltpu.VMEM((_PAD_H, _PAD_W), jnp.int32),
            pltpu.VMEM((_PAD_H, _PAD_W), jnp.int32),
            pltpu.SemaphoreType.DMA,
            pltpu.SemaphoreType.DMA,
        ],
        out_shape=jax.ShapeDtypeStruct((_SAMPLES, 1, 128), jnp.float32),
    )(label_image)

    nc = label_costs / jnp.maximum(jnp.sum(jnp.abs(label_costs)), _EPS)
    costs = []
    for s in range(_SAMPLES):
        hist = hist_pad[s, 0, :_NLAB]
        dist = hist / hist_pad[s, 0, _NLAB]
        costs.append(jnp.sum(nc * dist))
    best_cost = costs[0]
    best_idx = jnp.int32(0)
    for i in range(1, _SAMPLES):
        better = costs[i] < best_cost
        best_cost = jnp.where(better, costs[i], best_cost)
        best_idx = jnp.where(better, jnp.int32(i), best_idx)

    def _branch(s):
        def run(image, label_image):
            return pl.pallas_call(
                _make_extract_kernel(s),
                in_specs=[
                    pl.BlockSpec(memory_space=pl.ANY),
                    pl.BlockSpec(memory_space=pl.ANY),
                ],
                out_specs=[
                    pl.BlockSpec(memory_space=pltpu.VMEM),
                    pl.BlockSpec(memory_space=pltpu.VMEM),
                ],
                scratch_shapes=[
                    pltpu.VMEM((_PAD_H, _PAD_W), jnp.float32),
                    pltpu.VMEM((_PAD_H, _PAD_W), jnp.float32),
                    pltpu.VMEM((_PAD_H, _PAD_W), jnp.int32),
                    pltpu.SemaphoreType.DMA,
                    pltpu.SemaphoreType.DMA,
                    pltpu.SemaphoreType.DMA,
                ],
                out_shape=[
                    jax.ShapeDtypeStruct((3, _CROP_H, _CROP_W), jnp.float32),
                    jax.ShapeDtypeStruct((1, _CROP_H, _CROP_W), jnp.int32),
                ],
            )(image, label_image)

        return run

    best_image, best_label = jax.lax.switch(
        best_idx, [_branch(s) for s in range(_SAMPLES)], image, label_image
    )

    return best_image, best_label, best_cost


# speedup vs baseline: 11.4525x; 1.0132x over previous
import jax
import jax.numpy as jnp
from jax.experimental import pallas as pl
from jax.experimental.pallas import tpu as pltpu

_CROP_H, _CROP_W = 720, 1280
_H, _W = 2160, 3840
_SAMPLES = 4
_IGNORE = 0
_NLAB = 20
_EPS = 1e-12

_PAD_H = _CROP_H + 8
_PAD_W = _CROP_W + 128
_MAX_RS = _H - _PAD_H
_MAX_CS = _W - _PAD_W


def _crop_offsets():
    try:
        crop_key = jax.random.key(42)
        tops, lefts = [], []
        for i in range(_SAMPLES):
            kt, kl = jax.random.split(jax.random.fold_in(crop_key, i))
            tops.append(int(jax.random.randint(kt, (), 0, _H - _CROP_H + 1)))
            lefts.append(int(jax.random.randint(kl, (), 0, _W - _CROP_W + 1)))
        return tuple(tops), tuple(lefts)
    except Exception:
        return (564, 73, 133, 1175), (2217, 879, 2278, 255)


_TOPS, _LEFTS = _crop_offsets()


def _aligned(top, left):
    rs = min((top // 8) * 8, _MAX_RS)
    cs = min((left // 128) * 128, _MAX_CS)
    return rs, cs


def _hist_kernel(label_hbm, hist_ref, buf0, buf1, sem0, sem1):
    bufs = (buf0, buf1)
    sems = (sem0, sem1)

    def copy(s):
        rs, cs = _aligned(_TOPS[s], _LEFTS[s])
        return pltpu.make_async_copy(
            label_hbm.at[0, pl.ds(rs, _PAD_H), pl.ds(cs, _PAD_W)],
            bufs[s % 2],
            sems[s % 2],
        )

    copy(0).start()
    copy(1).start()
    for s in range(_SAMPLES):
        copy(s).wait()
        rs, cs = _aligned(_TOPS[s], _LEFTS[s])
        roff = _TOPS[s] - rs
        coff = _LEFTS[s] - cs
        labels = bufs[s % 2][roff : roff + _CROP_H, :]
        cols = jax.lax.broadcasted_iota(jnp.int32, (_CROP_H, _PAD_W), 1)
        inside = (cols >= coff) & (cols < coff + _CROP_W)
        labels = jnp.where(inside, labels, 0)
        sv = jnp.left_shift(jnp.int32(1), (labels & 3) * 8)
        g = labels >> 2
        parts = []
        for gi in range(5):
            contrib = jnp.where(g == gi, sv, 0)
            parts.append(contrib.reshape(3, 240, _PAD_W).sum(axis=1))
        lane = jax.lax.broadcasted_iota(jnp.int32, (1, 128), 1)
        vec = jnp.zeros((1, 128), jnp.int32)
        total = jnp.int32(0)
        for b in range(1, _NLAB):
            cnt = jnp.sum((parts[b >> 2] >> (8 * (b & 3))) & 255)
            vec = vec + jnp.where(lane == b, cnt, 0)
            total = total + cnt
        fvec = vec.astype(jnp.float32)
        Sv = jnp.maximum(total.astype(jnp.float32), jnp.float32(_EPS))
        fvec = fvec + jnp.where(lane == _NLAB, Sv, 0.0)
        hist_ref[s] = fvec
        if s + 2 < _SAMPLES:
            copy(s + 2).start()


def _make_extract_kernel(s):
    top, left = _TOPS[s], _LEFTS[s]
    rs, cs = _aligned(top, left)
    roff = top - rs
    coff = left - cs

    def _extract_kernel(img_hbm, lab_hbm, img_out, lab_out, fbuf0, fbuf1,
                        ibuf, obuf0, obuf1, olbuf,
                        sem0, sem1, sem_l, osem0, osem1, osem_l):

        def icopy(c, buf, sem):
            return pltpu.make_async_copy(
                img_hbm.at[c, pl.ds(rs, _PAD_H), pl.ds(cs, _PAD_W)], buf, sem
            )

        def window(buf):
            return buf[roff : roff + _CROP_H, coff : coff + _CROP_W]

        c0 = icopy(0, fbuf0, sem0)
        c1 = icopy(1, fbuf1, sem1)
        cl = pltpu.make_async_copy(
            lab_hbm.at[0, pl.ds(rs, _PAD_H), pl.ds(cs, _PAD_W)], ibuf, sem_l
        )
        c0.start()
        c1.start()
        cl.start()
        c0.wait()
        obuf0[...] = window(fbuf0)
        d0 = pltpu.make_async_copy(obuf0, img_out.at[0], osem0)
        d0.start()
        c1.wait()
        obuf1[...] = window(fbuf1)
        d1 = pltpu.make_async_copy(obuf1, img_out.at[1], osem1)
        d1.start()
        cl.wait()
        olbuf[...] = window(ibuf)
        dl = pltpu.make_async_copy(olbuf, lab_out.at[0], osem_l)
        dl.start()
        c2 = icopy(2, fbuf0, sem0)
        c2.start()
        c2.wait()
        d0.wait()
        obuf0[...] = window(fbuf0)
        d2 = pltpu.make_async_copy(obuf0, img_out.at[2], osem0)
        d2.start()
        d1.wait()
        dl.wait()
        d2.wait()

    return _extract_kernel


def kernel(image, label_image, label_costs):
    hist_pad = pl.pallas_call(
        _hist_kernel,
        in_specs=[pl.BlockSpec(memory_space=pl.ANY)],
        out_specs=pl.BlockSpec(memory_space=pltpu.VMEM),
        scratch_shapes=[
            pltpu.VMEM((_PAD_H, _PAD_W), jnp.int32),
            pltpu.VMEM((_PAD_H, _PAD_W), jnp.int32),
            pltpu.SemaphoreType.DMA,
            pltpu.SemaphoreType.DMA,
        ],
        out_shape=jax.ShapeDtypeStruct((_SAMPLES, 1, 128), jnp.float32),
    )(label_image)

    nc = label_costs / jnp.maximum(jnp.sum(jnp.abs(label_costs)), _EPS)
    costs = []
    for s in range(_SAMPLES):
        hist = hist_pad[s, 0, :_NLAB]
        dist = hist / hist_pad[s, 0, _NLAB]
        costs.append(jnp.sum(nc * dist))
    best_cost = costs[0]
    best_idx = jnp.int32(0)
    for i in range(1, _SAMPLES):
        better = costs[i] < best_cost
        best_cost = jnp.where(better, costs[i], best_cost)
        best_idx = jnp.where(better, jnp.int32(i), best_idx)

    def _branch(s):
        def run(image, label_image):
            return pl.pallas_call(
                _make_extract_kernel(s),
                in_specs=[
                    pl.BlockSpec(memory_space=pl.ANY),
                    pl.BlockSpec(memory_space=pl.ANY),
                ],
                out_specs=[
                    pl.BlockSpec(memory_space=pl.ANY),
                    pl.BlockSpec(memory_space=pl.ANY),
                ],
                scratch_shapes=[
                    pltpu.VMEM((_PAD_H, _PAD_W), jnp.float32),
                    pltpu.VMEM((_PAD_H, _PAD_W), jnp.float32),
                    pltpu.VMEM((_PAD_H, _PAD_W), jnp.int32),
                    pltpu.VMEM((_CROP_H, _CROP_W), jnp.float32),
                    pltpu.VMEM((_CROP_H, _CROP_W), jnp.float32),
                    pltpu.VMEM((_CROP_H, _CROP_W), jnp.int32),
                    pltpu.SemaphoreType.DMA,
                    pltpu.SemaphoreType.DMA,
                    pltpu.SemaphoreType.DMA,
                    pltpu.SemaphoreType.DMA,
                    pltpu.SemaphoreType.DMA,
                    pltpu.SemaphoreType.DMA,
                ],
                out_shape=[
                    jax.ShapeDtypeStruct((3, _CROP_H, _CROP_W), jnp.float32),
                    jax.ShapeDtypeStruct((1, _CROP_H, _CROP_W), jnp.int32),
                ],
            )(image, label_image)

        return run

    best_image, best_label = jax.lax.switch(
        best_idx, [_branch(s) for s in range(_SAMPLES)], image, label_image
    )

    return best_image, best_label, best_cost
